# baseline (speedup 1.0000x reference)
import functools

import jax
import jax.numpy as jnp
from jax import lax
from jax.experimental import pallas as pl
from jax.experimental.pallas import tpu as pltpu
from jax.experimental.pallas import tpu_sc as plsc

NC = 2
NS = 16
NW = NC * NS

LANES = 16
CHUNK = 128
NBUF = 4
ROUND_MAGIC = 8388608.0


@functools.lru_cache(maxsize=None)
def _build(b, t, a, vocab, embed_dim):
    n_total = b * t * a
    n_per_w = n_total // NW
    b_per_w = b // NW
    t_half = t // 2
    super_ = t_half * a
    n_super = n_per_w // super_
    gpc = -(-super_ // CHUNK)
    assert b_per_w * NW == b and t_half * 2 == t
    assert n_super * super_ == n_per_w
    assert n_super >= NBUF and (n_super - NBUF) % NBUF == 0
    assert super_ % LANES == 0
    mesh = plsc.VectorSubcoreMesh(core_axis_name="c", subcore_axis_name="s")

    @functools.partial(
        pl.kernel,
        mesh=mesh,
        compiler_params=pltpu.CompilerParams(use_tc_tiling_on_sc=False),
        out_type=jax.ShapeDtypeStruct((b, t, a, embed_dim), jnp.float32),
        scratch_types=[
            pltpu.VMEM((n_per_w,), jnp.float32),
            pltpu.VMEM((n_per_w,), jnp.int32),
            pltpu.VMEM((NBUF, super_, embed_dim), jnp.float32),
            pltpu.SemaphoreType.DMA,
            pltpu.SemaphoreType.DMA,
        ],
    )
    def k(act_hbm, table_hbm, out_hbm, act_v, idx_v, rows_v, sem_g, sem_s):
        wid = lax.axis_index("s") * NC + lax.axis_index("c")
        base = wid * n_per_w
        b0 = wid * b_per_w
        pltpu.sync_copy(act_hbm.at[pl.ds(base, n_per_w)], act_v)

        def fire(j, buf):
            for i in range(super_ // LANES):
                x = act_v[pl.ds(j * super_ + i * LANES, LANES)]
                s = (x - (-1.0)) / 2.0 * (vocab - 1.0)
                v = (s + ROUND_MAGIC) - ROUND_MAGIC
                v = jnp.minimum(jnp.maximum(v, 0.0), vocab - 1.0)
                idx_v[pl.ds(j * super_ + i * LANES, LANES)] = \
                    v.astype(jnp.int32)
            for g in range(gpc):
                lo = g * CHUNK
                sz = min(CHUNK, super_ - lo)
                pltpu.async_copy(
                    table_hbm.at[idx_v.at[pl.ds(j * super_ + lo, sz)]],
                    rows_v.at[buf].at[pl.ds(lo, sz)], sem_g)

        def wait_gathers():
            pltpu.make_async_copy(
                table_hbm.at[pl.ds(0, super_)], rows_v.at[0], sem_g).wait()

        def scatter(j, buf, half):
            bb = b0 + lax.div(j, 2)
            for tt in range(t_half):
                pltpu.async_copy(
                    rows_v.at[buf].at[pl.ds(tt * a, a)],
                    out_hbm.at[bb, half * t_half + tt], sem_s)

        def wait_one_scatter():
            for _ in range(t_half):
                pltpu.make_async_copy(
                    rows_v.at[0].at[pl.ds(0, a)], out_hbm.at[0, 0],
                    sem_s).wait()

        for j in range(NBUF - 1):
            fire(j, j)

        wait_gathers()
        scatter(0, 0, 0)
        fire(NBUF - 1, NBUF - 1)

        def body(i, _):
            for r in range(NBUF):
                j = 1 + NBUF * i + r
                wait_gathers()
                scatter(j, (1 + r) % NBUF, (1 + r) % 2)
                wait_one_scatter()
                fire(j + NBUF - 1, r)
            return 0

        lax.fori_loop(0, (n_super - NBUF) // NBUF, body, 0)

        for j in range(n_super - NBUF + 1, n_super):
            wait_gathers()
            scatter(j, j % NBUF, j % 2)
            wait_one_scatter()
        wait_one_scatter()

    return k


def kernel(actions, embedding):
    b, t, a = actions.shape
    vocab, embed_dim = embedding.shape
    return _build(b, t, a, vocab, embed_dim)(
        actions.reshape(b * t * a), embedding)


# speedup vs baseline: 1.4949x; 1.4949x over previous
import functools

import jax
import jax.numpy as jnp
from jax import lax
from jax.experimental import pallas as pl
from jax.experimental.pallas import tpu as pltpu
from jax.experimental.pallas import tpu_sc as plsc

NC = 2
NS = 16
NW = NC * NS

LANES = 16
NBUF = 4
ROUND_MAGIC = 8388608.0


@functools.lru_cache(maxsize=None)
def _build(b, t, a, vocab, embed_dim):
    n_pairs = t * a
    ppw = n_pairs // NW
    eh_n = embed_dim // 8
    bh_n = b // 128
    assert ppw * NW == n_pairs
    assert eh_n * 8 == embed_dim and bh_n * 128 == b
    assert eh_n % NBUF == 0 and b % LANES == 0
    mesh = plsc.VectorSubcoreMesh(core_axis_name="c", subcore_axis_name="s")

    @functools.partial(
        pl.kernel,
        mesh=mesh,
        compiler_params=pltpu.CompilerParams(use_tc_tiling_on_sc=False, needs_layout_passes=False),
        out_type=jax.ShapeDtypeStruct(
            (t, a, eh_n, bh_n, 8, 128), jnp.float32),
        scratch_types=[
            pltpu.VMEM((vocab * embed_dim,), jnp.float32),
            pltpu.VMEM((ppw * b,), jnp.float32),
            pltpu.VMEM((b,), jnp.int32),
            pltpu.VMEM((NBUF, bh_n, 8, 128), jnp.float32),
            pltpu.SemaphoreType.DMA,
        ],
    )
    def k(act_hbm, tab_hbm, out_hbm, tab_v, act_v, tok_v, bufs, sem_s):
        wid = lax.axis_index("s") * NC + lax.axis_index("c")
        p0 = wid * ppw
        pltpu.sync_copy(tab_hbm, tab_v)
        pltpu.sync_copy(act_hbm.at[pl.ds(p0 * b, ppw * b)], act_v)

        def wait_one_scatter():
            pltpu.make_async_copy(
                bufs.at[0], out_hbm.at[0, 0, 0], sem_s).wait()

        def pair_body(q, _):
            p = p0 + q
            tt = lax.div(p, a)
            aa = lax.rem(p, a)

            def quant(i, _):
                x = act_v[pl.ds(q * b + i * LANES, LANES)]
                s = (x - (-1.0)) / 2.0 * (vocab - 1.0)
                v = (s + ROUND_MAGIC) - ROUND_MAGIC
                v = jnp.minimum(jnp.maximum(v, 0.0), vocab - 1.0)
                tok_v[pl.ds(i * LANES, LANES)] = v.astype(jnp.int32)
                return 0

            lax.fori_loop(0, b // LANES, quant, 0, unroll=4)

            for eh in range(eh_n):
                buf = bufs.at[eh % NBUF]
                if eh < NBUF:
                    @pl.when(q > 0)
                    def _():
                        wait_one_scatter()
                else:
                    wait_one_scatter()

                def unit(bh, _):
                    for gg in range(8):
                        tok16 = tok_v[pl.ds(bh * 128 + gg * LANES, LANES)]
                        for el in range(8):
                            idx = tok16 + (eh * 8 + el) * vocab
                            val = plsc.load_gather(tab_v, [idx])
                            buf[bh, el, pl.ds(gg * LANES, LANES)] = val
                    return 0

                lax.fori_loop(0, bh_n, unit, 0)
                pltpu.async_copy(buf, out_hbm.at[tt, aa, eh], sem_s)
            return 0

        lax.fori_loop(0, ppw, pair_body, 0)
        for _ in range(NBUF):
            wait_one_scatter()

    return k


def kernel(actions, embedding):
    b, t, a = actions.shape
    vocab, embed_dim = embedding.shape
    act_flat = actions.transpose(1, 2, 0).reshape(b * t * a)
    tab_flat = embedding.T.reshape(vocab * embed_dim)
    o = _build(b, t, a, vocab, embed_dim)(act_flat, tab_flat)
    o = o.transpose(3, 5, 0, 1, 2, 4)
    return o.reshape(b, t, a, embed_dim)


# speedup vs baseline: 6.2022x; 4.1488x over previous
import functools

import jax
import jax.numpy as jnp
from jax import lax
from jax.experimental import pallas as pl
from jax.experimental.pallas import tpu as pltpu
from jax.experimental.pallas import tpu_sc as plsc

NC = 2
NS = 16
NW = NC * NS

LANES = 16
NBUF = 4
ROUND_MAGIC = 8388608.0


@functools.lru_cache(maxsize=None)
def _build(b, t, a, vocab, embed_dim):
    n_pairs = t * a
    ppw = n_pairs // NW
    eh_n = embed_dim // 8
    bh_n = b // 128
    assert ppw * NW == n_pairs
    assert eh_n * 8 == embed_dim and bh_n * 128 == b
    assert eh_n % NBUF == 0 and b % LANES == 0
    mesh = plsc.VectorSubcoreMesh(core_axis_name="c", subcore_axis_name="s")

    @functools.partial(
        pl.kernel,
        mesh=mesh,
        compiler_params=pltpu.CompilerParams(use_tc_tiling_on_sc=False, needs_layout_passes=False),
        out_type=jax.ShapeDtypeStruct(
            (t, a, eh_n, bh_n, 8, 128), jnp.float32),
        scratch_types=[
            pltpu.VMEM((vocab * embed_dim,), jnp.float32),
            pltpu.VMEM((ppw * b,), jnp.float32),
            pltpu.VMEM((b,), jnp.int32),
            pltpu.VMEM((NBUF, bh_n, 8, 128), jnp.float32),
            pltpu.SemaphoreType.DMA,
        ],
    )
    def k(act_hbm, tab_hbm, out_hbm, tab_v, act_v, tok_v, bufs, sem_s):
        wid = lax.axis_index("s") * NC + lax.axis_index("c")
        p0 = wid * ppw
        pltpu.sync_copy(tab_hbm, tab_v)
        pltpu.sync_copy(act_hbm.at[pl.ds(p0 * b, ppw * b)], act_v)

        def wait_one_scatter():
            pltpu.make_async_copy(
                bufs.at[0], out_hbm.at[0, 0, 0], sem_s).wait()

        def pair_body(q, _):
            p = p0 + q
            tt = lax.div(p, a)
            aa = lax.rem(p, a)

            @plsc.parallel_loop(0, b // LANES, unroll=4)
            def quant(i):
                x = act_v[pl.ds(q * b + i * LANES, LANES)]
                s = (x - (-1.0)) / 2.0 * (vocab - 1.0)
                v = (s + ROUND_MAGIC) - ROUND_MAGIC
                v = jnp.minimum(jnp.maximum(v, 0.0), vocab - 1.0)
                tok_v[pl.ds(i * LANES, LANES)] = v.astype(jnp.int32)

            for eh in range(eh_n):
                buf = bufs.at[eh % NBUF]
                if eh < NBUF:
                    @pl.when(q > 0)
                    def _():
                        wait_one_scatter()
                else:
                    wait_one_scatter()

                @plsc.parallel_loop(0, bh_n * 8, unroll=2)
                def unit(g):
                    bh = lax.div(g, 8)
                    gg = lax.rem(g, 8)
                    tok16 = tok_v[pl.ds(g * LANES, LANES)]
                    for el in range(8):
                        idx = tok16 + (eh * 8 + el) * vocab
                        val = plsc.load_gather(tab_v, [idx])
                        buf[bh, el, pl.ds(gg * LANES, LANES)] = val
                pltpu.async_copy(buf, out_hbm.at[tt, aa, eh], sem_s)
            return 0

        lax.fori_loop(0, ppw, pair_body, 0)
        for _ in range(NBUF):
            wait_one_scatter()

    return k


def kernel(actions, embedding):
    b, t, a = actions.shape
    vocab, embed_dim = embedding.shape
    act_flat = actions.transpose(1, 2, 0).reshape(b * t * a)
    tab_flat = embedding.T.reshape(vocab * embed_dim)
    o = _build(b, t, a, vocab, embed_dim)(act_flat, tab_flat)
    o = o.transpose(3, 5, 0, 1, 2, 4)
    return o.reshape(b, t, a, embed_dim)
